# baseline (speedup 1.0000x reference)
import functools

import jax
import jax.numpy as jnp
from jax.experimental import pallas as pl
from jax.experimental.pallas import tpu as pltpu


def _feature_kernel(v_ref, w_ref, x_ref):
    x_ref[...] = jnp.dot(v_ref[...], w_ref[...],
                         preferred_element_type=jnp.float32)


def _spmm_kernel(adj_ref, x_ref, bias_ref, out_ref, *, out_f):
    adj = adj_ref[...]
    xs = x_ref[...]
    a1 = (adj == 1).astype(jnp.float32)
    a2 = (adj == 2).astype(jnp.float32)
    a3 = (adj == 3).astype(jnp.float32)
    acc = jnp.dot(a1, xs[:, :out_f], preferred_element_type=jnp.float32)
    acc += jnp.dot(a2, xs[:, out_f:2 * out_f],
                   preferred_element_type=jnp.float32)
    acc += jnp.dot(a3, xs[:, 2 * out_f:],
                   preferred_element_type=jnp.float32)
    out_ref[...] = acc + bias_ref[...]


def kernel(V, adj, w1, w2, w3, bias):
    n, in_f = V.shape
    out_f = w1.shape[1]
    w = jnp.concatenate([w1, w2, w3], axis=1)

    bm_x = 1024
    x = pl.pallas_call(
        _feature_kernel,
        grid=(n // bm_x,),
        in_specs=[
            pl.BlockSpec((bm_x, in_f), lambda i: (i, 0)),
            pl.BlockSpec((in_f, 3 * out_f), lambda i: (0, 0)),
        ],
        out_specs=pl.BlockSpec((bm_x, 3 * out_f), lambda i: (i, 0)),
        out_shape=jax.ShapeDtypeStruct((n, 3 * out_f), jnp.float32),
    )(V, w)

    bm = 128
    body = functools.partial(_spmm_kernel, out_f=out_f)
    out = pl.pallas_call(
        body,
        grid=(n // bm,),
        in_specs=[
            pl.BlockSpec((bm, n), lambda i: (i, 0)),
            pl.BlockSpec((n, 3 * out_f), lambda i: (0, 0)),
            pl.BlockSpec((1, out_f), lambda i: (0, 0)),
        ],
        out_specs=pl.BlockSpec((bm, out_f), lambda i: (i, 0)),
        out_shape=jax.ShapeDtypeStruct((n, out_f), jnp.float32),
        compiler_params=pltpu.CompilerParams(
            dimension_semantics=("parallel",),
        ),
    )(adj, x, bias.reshape(1, out_f))
    return out


# speedup vs baseline: 1.1199x; 1.1199x over previous
import functools

import jax
import jax.numpy as jnp
from jax.experimental import pallas as pl
from jax.experimental.pallas import tpu as pltpu


def _feature_kernel(v_ref, w_ref, x_ref):
    x_ref[...] = jnp.dot(v_ref[...], w_ref[...],
                         preferred_element_type=jnp.float32)


def _outer_kernel(adj_hbm, x_ref, bias_ref, out_ref, *, n, bm, out_f):
    def body(adj_blk, out_blk):
        adj = adj_blk[...]
        xs = x_ref[...]
        a1 = (adj == 1).astype(jnp.float32)
        a2 = (adj == 2).astype(jnp.float32)
        a3 = (adj == 3).astype(jnp.float32)
        acc = jnp.dot(a1, xs[:, :out_f], preferred_element_type=jnp.float32)
        acc += jnp.dot(a2, xs[:, out_f:2 * out_f],
                       preferred_element_type=jnp.float32)
        acc += jnp.dot(a3, xs[:, 2 * out_f:],
                       preferred_element_type=jnp.float32)
        out_blk[...] = acc + bias_ref[...]

    pltpu.emit_pipeline(
        body,
        grid=(n // bm,),
        in_specs=[
            pl.BlockSpec((bm, n), lambda i: (i, 0),
                         pipeline_mode=pl.Buffered(buffer_count=3)),
        ],
        out_specs=[pl.BlockSpec((bm, out_f), lambda i: (i, 0))],
    )(adj_hbm, out_ref)


def kernel(V, adj, w1, w2, w3, bias):
    n, in_f = V.shape
    out_f = w1.shape[1]
    w = jnp.concatenate([w1, w2, w3], axis=1)

    bm_x = 1024
    x = pl.pallas_call(
        _feature_kernel,
        grid=(n // bm_x,),
        in_specs=[
            pl.BlockSpec((bm_x, in_f), lambda i: (i, 0)),
            pl.BlockSpec((in_f, 3 * out_f), lambda i: (0, 0)),
        ],
        out_specs=pl.BlockSpec((bm_x, 3 * out_f), lambda i: (i, 0)),
        out_shape=jax.ShapeDtypeStruct((n, 3 * out_f), jnp.float32),
    )(V, w)

    bm = 512
    body = functools.partial(_outer_kernel, n=n, bm=bm, out_f=out_f)
    out = pl.pallas_call(
        body,
        in_specs=[
            pl.BlockSpec(memory_space=pl.ANY),
            pl.BlockSpec(memory_space=pltpu.VMEM),
            pl.BlockSpec(memory_space=pltpu.VMEM),
        ],
        out_specs=pl.BlockSpec(memory_space=pl.ANY),
        out_shape=jax.ShapeDtypeStruct((n, out_f), jnp.float32),
        compiler_params=pltpu.CompilerParams(
            vmem_limit_bytes=66_000_000,
        ),
    )(adj, x, bias.reshape(1, out_f))
    return out


# speedup vs baseline: 1.1896x; 1.0622x over previous
import functools

import jax
import jax.numpy as jnp
from jax.experimental import pallas as pl
from jax.experimental.pallas import tpu as pltpu


def _feature_kernel(v_ref, w_ref, x_ref):
    x_ref[...] = jnp.dot(v_ref[...], w_ref[...],
                         preferred_element_type=jnp.float32
                         ).astype(jnp.bfloat16)


def _spmm_kernel(adj_ref, x_ref, bias_ref, out_ref, *, out_f):
    adj = adj_ref[...]
    xs = x_ref[...]
    a1 = (adj == 1).astype(jnp.float32)
    a2 = (adj == 2).astype(jnp.float32)
    a3 = (adj == 3).astype(jnp.float32)
    dn = (((1,), (0,)), ((), ()))
    acc = jax.lax.dot_general(a1, xs[:, :out_f], dn,
                              preferred_element_type=jnp.float32)
    acc += jax.lax.dot_general(a2, xs[:, out_f:2 * out_f], dn,
                               preferred_element_type=jnp.float32)
    acc += jax.lax.dot_general(a3, xs[:, 2 * out_f:], dn,
                               preferred_element_type=jnp.float32)
    out_ref[...] = acc + bias_ref[...]


def kernel(V, adj, w1, w2, w3, bias):
    n, in_f = V.shape
    out_f = w1.shape[1]
    w = jnp.concatenate([w1, w2, w3], axis=1)

    bm_x = 1024
    x = pl.pallas_call(
        _feature_kernel,
        grid=(n // bm_x,),
        in_specs=[
            pl.BlockSpec((bm_x, in_f), lambda i: (i, 0)),
            pl.BlockSpec((in_f, 3 * out_f), lambda i: (0, 0)),
        ],
        out_specs=pl.BlockSpec((bm_x, 3 * out_f), lambda i: (i, 0)),
        out_shape=jax.ShapeDtypeStruct((n, 3 * out_f), jnp.bfloat16),
    )(V, w)

    bm = 512
    body = functools.partial(_spmm_kernel, out_f=out_f)
    out = pl.pallas_call(
        body,
        grid=(n // bm,),
        in_specs=[
            pl.BlockSpec((bm, n), lambda i: (i, 0)),
            pl.BlockSpec((n, 3 * out_f), lambda i: (0, 0)),
            pl.BlockSpec((1, out_f), lambda i: (0, 0)),
        ],
        out_specs=pl.BlockSpec((bm, out_f), lambda i: (i, 0)),
        out_shape=jax.ShapeDtypeStruct((n, out_f), jnp.float32),
        compiler_params=pltpu.CompilerParams(
            dimension_semantics=("parallel",),
        ),
    )(adj, x, bias.reshape(1, out_f))
    return out


# speedup vs baseline: 1.2603x; 1.0594x over previous
import functools

import jax
import jax.numpy as jnp
from jax.experimental import pallas as pl
from jax.experimental.pallas import tpu as pltpu


def _fused_kernel(v_ref, w_ref, adj_ref, bias_ref, out_ref, x_scratch, *,
                  out_f):
    s = pl.program_id(0)

    @pl.when(s == 0)
    def _features():
        x_scratch[...] = jnp.dot(v_ref[...], w_ref[...],
                                 preferred_element_type=jnp.float32
                                 ).astype(jnp.bfloat16)

    @pl.when(s > 0)
    def _spmm():
        adj = adj_ref[...]
        xs = x_scratch[...]
        a1 = (adj == 1).astype(jnp.float32)
        a2 = (adj == 2).astype(jnp.float32)
        a3 = (adj == 3).astype(jnp.float32)
        dn = (((1,), (0,)), ((), ()))
        acc = jax.lax.dot_general(a1, xs[:, :out_f], dn,
                                  preferred_element_type=jnp.float32)
        acc += jax.lax.dot_general(a2, xs[:, out_f:2 * out_f], dn,
                                   preferred_element_type=jnp.float32)
        acc += jax.lax.dot_general(a3, xs[:, 2 * out_f:], dn,
                                   preferred_element_type=jnp.float32)
        out_ref[...] = acc + bias_ref[...]


def kernel(V, adj, w1, w2, w3, bias):
    n, in_f = V.shape
    out_f = w1.shape[1]
    w = jnp.concatenate([w1, w2, w3], axis=1)

    bm = 512
    body = functools.partial(_fused_kernel, out_f=out_f)
    out = pl.pallas_call(
        body,
        grid=(n // bm + 1,),
        in_specs=[
            pl.BlockSpec((n, in_f), lambda s: (0, 0)),
            pl.BlockSpec((in_f, 3 * out_f), lambda s: (0, 0)),
            pl.BlockSpec((bm, n), lambda s: (jnp.maximum(s - 1, 0), 0)),
            pl.BlockSpec((1, out_f), lambda s: (0, 0)),
        ],
        out_specs=pl.BlockSpec((bm, out_f),
                               lambda s: (jnp.maximum(s - 1, 0), 0)),
        out_shape=jax.ShapeDtypeStruct((n, out_f), jnp.float32),
        scratch_shapes=[pltpu.VMEM((n, 3 * out_f), jnp.bfloat16)],
        compiler_params=pltpu.CompilerParams(
            dimension_semantics=("arbitrary",),
        ),
    )(V, w, adj, bias.reshape(1, out_f))
    return out
